# baseline (speedup 1.0000x reference)
import functools

import jax
import jax.numpy as jnp
from jax import lax
from jax.experimental import pallas as pl
from jax.experimental.pallas import tpu as pltpu
from jax.experimental.pallas import tpu_sc as plsc

N_TOK = 8192
D = 1024
E = 16
K = 2

BM = 128
S = N_TOK * K
S_PAD = S + E * BM
T_TILES = S_PAD // BM

BT = 512
NBT = N_TOK // BT

NC, NS, L = 2, 16, 16
NW = NC * NS
R_TOK = N_TOK // NW

CD = 32
NCH_D = R_TOK // CD

C2 = 16
NCH_C = R_TOK // C2
VG = D // (4 * L)


def _router_body(x_ref, w_ref, b_ref, idx_ref, gates_ref, ranks_ref,
                 counts_ref, carry_ref):
    i = pl.program_id(0)

    @pl.when(i == 0)
    def _():
        carry_ref[...] = jnp.zeros_like(carry_ref)

    xb = x_ref[...]
    logits = jnp.dot(xb, w_ref[...], preferred_element_type=jnp.float32)
    logits = logits + b_ref[0:1, :]

    iota_e = lax.broadcasted_iota(jnp.int32, (BT, E), 1)
    m1 = jnp.max(logits, axis=1, keepdims=True)
    i1 = jnp.min(jnp.where(logits == m1, iota_e, E), axis=1, keepdims=True)
    masked = jnp.where(iota_e == i1, -1e30, logits)
    m2 = jnp.max(masked, axis=1, keepdims=True)
    i2 = jnp.min(jnp.where(masked == m2, iota_e, E), axis=1, keepdims=True)
    g1 = 1.0 / (1.0 + jnp.exp(m2 - m1))
    g2 = 1.0 - g1

    onehot = (jnp.where(iota_e == i1, 1.0, 0.0)
              + jnp.where(iota_e == i2, 1.0, 0.0))
    row_i = lax.broadcasted_iota(jnp.int32, (BT, BT), 0)
    col_i = lax.broadcasted_iota(jnp.int32, (BT, BT), 1)
    tri = jnp.where(row_i > col_i, 1.0, 0.0)
    csum = jnp.dot(tri, onehot, preferred_element_type=jnp.float32)
    ranks_mat = csum + carry_ref[0:1, :]
    r1 = jnp.sum(jnp.where(iota_e == i1, ranks_mat, 0.0), axis=1, keepdims=True)
    r2 = jnp.sum(jnp.where(iota_e == i2, ranks_mat, 0.0), axis=1, keepdims=True)

    carry_new = carry_ref[0:1, :] + jnp.sum(onehot, axis=0, keepdims=True)
    carry_ref[0:1, :] = carry_new

    idx_ref[...] = jnp.concatenate([i1, i2], axis=1)
    gates_ref[...] = jnp.concatenate([g1, g2], axis=1)
    ranks_ref[...] = jnp.concatenate([r1, r2], axis=1).astype(jnp.int32)
    counts_ref[...] = jnp.broadcast_to(carry_new, (8, E))


def _run_router(x, router_w, router_b):
    b2d = jnp.broadcast_to(router_b[None, :], (8, E))
    return pl.pallas_call(
        _router_body,
        grid=(NBT,),
        in_specs=[
            pl.BlockSpec((BT, D), lambda i: (i, 0)),
            pl.BlockSpec((D, E), lambda i: (0, 0)),
            pl.BlockSpec((8, E), lambda i: (0, 0)),
        ],
        out_specs=[
            pl.BlockSpec((BT, K), lambda i: (i, 0)),
            pl.BlockSpec((BT, K), lambda i: (i, 0)),
            pl.BlockSpec((BT, K), lambda i: (i, 0)),
            pl.BlockSpec((8, E), lambda i: (0, 0)),
        ],
        out_shape=[
            jax.ShapeDtypeStruct((N_TOK, K), jnp.int32),
            jax.ShapeDtypeStruct((N_TOK, K), jnp.float32),
            jax.ShapeDtypeStruct((N_TOK, K), jnp.int32),
            jax.ShapeDtypeStruct((8, E), jnp.float32),
        ],
        scratch_shapes=[pltpu.VMEM((8, E), jnp.float32)],
    )(x, router_w, b2d)


def _dispatch_body(x_hbm, pos_hbm, xpad_hbm, p0_v, p1_v, buf0, buf1,
                   rsem0, rsem1, ssem0, ssem1):
    wid = lax.axis_index("s") * NC + lax.axis_index("c")
    base = wid * R_TOK
    pltpu.sync_copy(pos_hbm.at[0, wid], p0_v)
    pltpu.sync_copy(pos_hbm.at[1, wid], p1_v)
    bufs = (buf0, buf1)
    rsems = (rsem0, rsem1)
    ssems = (ssem0, ssem1)
    reads = {0: pltpu.async_copy(x_hbm.at[pl.ds(base, CD)], buf0, rsem0)}
    scats = {}
    for j in range(NCH_D):
        reads[j].wait()
        b = bufs[j % 2]
        s = ssems[j % 2]
        scats[j] = (pltpu.async_copy(b, xpad_hbm.at[p0_v.at[j]], s),
                    pltpu.async_copy(b, xpad_hbm.at[p1_v.at[j]], s))
        if j + 1 < NCH_D:
            if j >= 1:
                scats[j - 1][0].wait()
                scats[j - 1][1].wait()
            reads[j + 1] = pltpu.async_copy(
                x_hbm.at[pl.ds(base + (j + 1) * CD, CD)],
                bufs[(j + 1) % 2], rsems[(j + 1) % 2])
    scats[NCH_D - 2][0].wait()
    scats[NCH_D - 2][1].wait()
    scats[NCH_D - 1][0].wait()
    scats[NCH_D - 1][1].wait()


@functools.cache
def _dispatch():
    return pl.kernel(
        _dispatch_body,
        out_type=jax.ShapeDtypeStruct((S_PAD, D), jnp.float32),
        mesh=plsc.VectorSubcoreMesh(core_axis_name="c", subcore_axis_name="s"),
        scratch_types=[
            pltpu.VMEM((NCH_D, CD), jnp.int32),
            pltpu.VMEM((NCH_D, CD), jnp.int32),
            pltpu.VMEM((CD, D), jnp.float32),
            pltpu.VMEM((CD, D), jnp.float32),
            pltpu.SemaphoreType.DMA,
            pltpu.SemaphoreType.DMA,
            pltpu.SemaphoreType.DMA,
            pltpu.SemaphoreType.DMA,
        ],
    )


def _gmm_body(te_ref, x_ref, w_ref, b_ref, o_ref):
    yb = jnp.dot(x_ref[...], w_ref[0], preferred_element_type=jnp.float32)
    o_ref[...] = yb + b_ref[0]


def _run_gmm(x_pad, expert_w, expert_b, tile_expert):
    grid_spec = pltpu.PrefetchScalarGridSpec(
        num_scalar_prefetch=1,
        grid=(T_TILES,),
        in_specs=[
            pl.BlockSpec((BM, D), lambda i, te: (i, 0)),
            pl.BlockSpec((1, D, D), lambda i, te: (te[i], 0, 0)),
            pl.BlockSpec((1, 1, D), lambda i, te: (te[i], 0, 0)),
        ],
        out_specs=pl.BlockSpec((BM, D), lambda i, te: (i, 0)),
    )
    return pl.pallas_call(
        _gmm_body,
        grid_spec=grid_spec,
        out_shape=jax.ShapeDtypeStruct((S_PAD, D), jnp.float32),
    )(tile_expert, x_pad, expert_w, expert_b.reshape(E, 1, D))


def _combine_body(ypad_hbm, pos_hbm, gate_hbm, out_hbm, p0_v, p1_v, g0_v, g1_v,
                  a0, a1, b0, b1, gsemA, gsemB, wsemA, wsemB):
    wid = lax.axis_index("s") * NC + lax.axis_index("c")
    base = wid * R_TOK
    pltpu.sync_copy(pos_hbm.at[0, wid], p0_v)
    pltpu.sync_copy(pos_hbm.at[1, wid], p1_v)
    pltpu.sync_copy(gate_hbm.at[0, wid], g0_v)
    pltpu.sync_copy(gate_hbm.at[1, wid], g1_v)
    pairs = ((a0, a1, gsemA, wsemA), (b0, b1, gsemB, wsemB))
    gath = {0: (pltpu.async_copy(ypad_hbm.at[p0_v.at[0]], a0, gsemA),
                pltpu.async_copy(ypad_hbm.at[p1_v.at[0]], a1, gsemA))}
    writes = {}
    for j in range(NCH_C):
        c0, c1, gsem, wsem = pairs[j % 2]
        gath[j][0].wait()
        gath[j][1].wait()
        if j + 1 < NCH_C:
            n0, n1, ngsem, nwsem = pairs[(j + 1) % 2]
            if j >= 1:
                writes[j - 1].wait()
            gath[j + 1] = (
                pltpu.async_copy(ypad_hbm.at[p0_v.at[j + 1]], n0, ngsem),
                pltpu.async_copy(ypad_hbm.at[p1_v.at[j + 1]], n1, ngsem))

        def row_body(r, _):
            tid = j * C2 + r
            gsl = pl.ds((tid % 8) * L, L)
            gv0 = g0_v[tid // 8, gsl]
            gv1 = g1_v[tid // 8, gsl]

            def vec_body(v, _):
                for u in range(4):
                    sl = pl.ds(v * (4 * L) + u * L, L)
                    c0[r, sl] = c0[r, sl] * gv0 + c1[r, sl] * gv1
                return 0

            lax.fori_loop(0, VG, vec_body, 0)
            return 0

        lax.fori_loop(0, C2, row_body, 0)
        writes[j] = pltpu.async_copy(
            c0, out_hbm.at[pl.ds(base + j * C2, C2)], wsem)
    writes[NCH_C - 2].wait()
    writes[NCH_C - 1].wait()


@functools.cache
def _combine():
    return pl.kernel(
        _combine_body,
        out_type=jax.ShapeDtypeStruct((N_TOK, D), jnp.float32),
        mesh=plsc.VectorSubcoreMesh(core_axis_name="c", subcore_axis_name="s"),
        scratch_types=[
            pltpu.VMEM((NCH_C, C2), jnp.int32),
            pltpu.VMEM((NCH_C, C2), jnp.int32),
            pltpu.VMEM((R_TOK // 8, 8 * L), jnp.float32),
            pltpu.VMEM((R_TOK // 8, 8 * L), jnp.float32),
            pltpu.VMEM((C2, D), jnp.float32),
            pltpu.VMEM((C2, D), jnp.float32),
            pltpu.VMEM((C2, D), jnp.float32),
            pltpu.VMEM((C2, D), jnp.float32),
            pltpu.SemaphoreType.DMA,
            pltpu.SemaphoreType.DMA,
            pltpu.SemaphoreType.DMA,
            pltpu.SemaphoreType.DMA,
        ],
    )


def kernel(x, router_w, router_b, expert_w, expert_b):
    idx, gates, ranks, counts8 = _run_router(x, router_w, router_b)
    counts = counts8[0].astype(jnp.int32)

    gp = ((counts + BM - 1) // BM) * BM
    cgp = jnp.cumsum(gp)
    off = jnp.concatenate([jnp.zeros((1,), jnp.int32), cgp])
    pos = off[idx] + ranks
    tile_starts = jnp.arange(T_TILES, dtype=jnp.int32)[:, None] * BM
    tile_expert = jnp.minimum(
        jnp.sum(jnp.where(tile_starts >= cgp[None, :], 1, 0), axis=1),
        E - 1).astype(jnp.int32)

    pos_t = pos.T.reshape(K, NW, NCH_D, CD)
    x_pad = _dispatch()(x, pos_t)
    y_pad = _run_gmm(x_pad, expert_w, expert_b, tile_expert)
    pos_c = pos.T.reshape(K, NW, NCH_C, C2)
    gate_b = jnp.broadcast_to(
        gates.T.reshape(K, NW, R_TOK, 1),
        (K, NW, R_TOK, L)).reshape(K, NW, R_TOK // 8, 8 * L)
    return _combine()(y_pad, pos_c, gate_b)


# speedup vs baseline: 1.0738x; 1.0738x over previous
import functools

import jax
import jax.numpy as jnp
from jax import lax
from jax.experimental import pallas as pl
from jax.experimental.pallas import tpu as pltpu
from jax.experimental.pallas import tpu_sc as plsc

N_TOK = 8192
D = 1024
E = 16
K = 2

BM = 256
S = N_TOK * K
S_PAD = S + E * BM
T_TILES = S_PAD // BM

BT = 512
NBT = N_TOK // BT

NC, NS, L = 2, 16, 16
NW = NC * NS
R_TOK = N_TOK // NW

CD = 32
NCH_D = R_TOK // CD

C2 = 16
NCH_C = R_TOK // C2
VG = D // (4 * L)


def _router_body(x_ref, w_ref, b_ref, idx_ref, gates_ref, ranks_ref,
                 counts_ref, carry_ref):
    i = pl.program_id(0)

    @pl.when(i == 0)
    def _():
        carry_ref[...] = jnp.zeros_like(carry_ref)

    xb = x_ref[...]
    logits = jnp.dot(xb, w_ref[...], preferred_element_type=jnp.float32)
    logits = logits + b_ref[0:1, :]

    iota_e = lax.broadcasted_iota(jnp.int32, (BT, E), 1)
    m1 = jnp.max(logits, axis=1, keepdims=True)
    i1 = jnp.min(jnp.where(logits == m1, iota_e, E), axis=1, keepdims=True)
    masked = jnp.where(iota_e == i1, -1e30, logits)
    m2 = jnp.max(masked, axis=1, keepdims=True)
    i2 = jnp.min(jnp.where(masked == m2, iota_e, E), axis=1, keepdims=True)
    g1 = 1.0 / (1.0 + jnp.exp(m2 - m1))
    g2 = 1.0 - g1

    onehot = (jnp.where(iota_e == i1, 1.0, 0.0)
              + jnp.where(iota_e == i2, 1.0, 0.0))
    row_i = lax.broadcasted_iota(jnp.int32, (BT, BT), 0)
    col_i = lax.broadcasted_iota(jnp.int32, (BT, BT), 1)
    tri = jnp.where(row_i > col_i, 1.0, 0.0)
    csum = jnp.dot(tri, onehot, preferred_element_type=jnp.float32)
    ranks_mat = csum + carry_ref[0:1, :]
    r1 = jnp.sum(jnp.where(iota_e == i1, ranks_mat, 0.0), axis=1, keepdims=True)
    r2 = jnp.sum(jnp.where(iota_e == i2, ranks_mat, 0.0), axis=1, keepdims=True)

    carry_new = carry_ref[0:1, :] + jnp.sum(onehot, axis=0, keepdims=True)
    carry_ref[0:1, :] = carry_new

    idx_ref[...] = jnp.concatenate([i1, i2], axis=1)
    gates_ref[...] = jnp.concatenate([g1, g2], axis=1)
    ranks_ref[...] = jnp.concatenate([r1, r2], axis=1).astype(jnp.int32)
    counts_ref[...] = jnp.broadcast_to(carry_new, (8, E))


def _run_router(x, router_w, router_b):
    b2d = jnp.broadcast_to(router_b[None, :], (8, E))
    return pl.pallas_call(
        _router_body,
        grid=(NBT,),
        in_specs=[
            pl.BlockSpec((BT, D), lambda i: (i, 0)),
            pl.BlockSpec((D, E), lambda i: (0, 0)),
            pl.BlockSpec((8, E), lambda i: (0, 0)),
        ],
        out_specs=[
            pl.BlockSpec((BT, K), lambda i: (i, 0)),
            pl.BlockSpec((BT, K), lambda i: (i, 0)),
            pl.BlockSpec((BT, K), lambda i: (i, 0)),
            pl.BlockSpec((8, E), lambda i: (0, 0)),
        ],
        out_shape=[
            jax.ShapeDtypeStruct((N_TOK, K), jnp.int32),
            jax.ShapeDtypeStruct((N_TOK, K), jnp.float32),
            jax.ShapeDtypeStruct((N_TOK, K), jnp.int32),
            jax.ShapeDtypeStruct((8, E), jnp.float32),
        ],
        scratch_shapes=[pltpu.VMEM((8, E), jnp.float32)],
    )(x, router_w, b2d)


def _fill_pos(eix_hbm, rnk_hbm, wid, off_ref, e_v, r_v, p_v, nch, cw):
    pltpu.sync_copy(eix_hbm.at[wid], e_v)
    pltpu.sync_copy(rnk_hbm.at[wid], r_v)
    off = off_ref[...]
    vecs_per_chunk = cw // L
    for v in range(R_TOK // L):
        pos = off.at[e_v[v, :]].get(mode="promise_in_bounds") + r_v[v, :]
        p_v[v // vecs_per_chunk,
            pl.ds((v % vecs_per_chunk) * L, L)] = pos


def _dispatch_body(x_hbm, e0_hbm, e1_hbm, r0_hbm, r1_hbm, off_hbm,
                   xpad_hbm, off_ref, e_v, r_v, p0_v, p1_v, buf0, buf1,
                   rsem0, rsem1, ssem0, ssem1):
    wid = lax.axis_index("s") * NC + lax.axis_index("c")
    base = wid * R_TOK
    reads = {0: pltpu.async_copy(x_hbm.at[pl.ds(base, CD)], buf0, rsem0)}
    pltpu.sync_copy(off_hbm.at[0], off_ref)
    _fill_pos(e0_hbm, r0_hbm, wid, off_ref, e_v, r_v, p0_v, NCH_D, CD)
    _fill_pos(e1_hbm, r1_hbm, wid, off_ref, e_v, r_v, p1_v, NCH_D, CD)
    bufs = (buf0, buf1)
    rsems = (rsem0, rsem1)
    ssems = (ssem0, ssem1)
    scats = {}
    for j in range(NCH_D):
        reads[j].wait()
        b = bufs[j % 2]
        s = ssems[j % 2]
        scats[j] = (pltpu.async_copy(b, xpad_hbm.at[p0_v.at[j]], s),
                    pltpu.async_copy(b, xpad_hbm.at[p1_v.at[j]], s))
        if j + 1 < NCH_D:
            if j >= 1:
                scats[j - 1][0].wait()
                scats[j - 1][1].wait()
            reads[j + 1] = pltpu.async_copy(
                x_hbm.at[pl.ds(base + (j + 1) * CD, CD)],
                bufs[(j + 1) % 2], rsems[(j + 1) % 2])
    scats[NCH_D - 2][0].wait()
    scats[NCH_D - 2][1].wait()
    scats[NCH_D - 1][0].wait()
    scats[NCH_D - 1][1].wait()


@functools.cache
def _dispatch():
    return pl.kernel(
        _dispatch_body,
        out_type=jax.ShapeDtypeStruct((S_PAD, D), jnp.float32),
        mesh=plsc.VectorSubcoreMesh(core_axis_name="c", subcore_axis_name="s"),
        scratch_types=[
            pltpu.VMEM((L,), jnp.int32),
            pltpu.VMEM((R_TOK // L, L), jnp.int32),
            pltpu.VMEM((R_TOK // L, L), jnp.int32),
            pltpu.VMEM((NCH_D, CD), jnp.int32),
            pltpu.VMEM((NCH_D, CD), jnp.int32),
            pltpu.VMEM((CD, D), jnp.float32),
            pltpu.VMEM((CD, D), jnp.float32),
            pltpu.SemaphoreType.DMA,
            pltpu.SemaphoreType.DMA,
            pltpu.SemaphoreType.DMA,
            pltpu.SemaphoreType.DMA,
        ],
    )


def _gmm_body(te_ref, x_ref, w_ref, b_ref, o_ref):
    yb = jnp.dot(x_ref[...], w_ref[0], preferred_element_type=jnp.float32)
    o_ref[...] = yb + b_ref[0]


def _run_gmm(x_pad, expert_w, expert_b, tile_expert):
    grid_spec = pltpu.PrefetchScalarGridSpec(
        num_scalar_prefetch=1,
        grid=(T_TILES,),
        in_specs=[
            pl.BlockSpec((BM, D), lambda i, te: (i, 0)),
            pl.BlockSpec((1, D, D), lambda i, te: (te[i], 0, 0)),
            pl.BlockSpec((1, 1, D), lambda i, te: (te[i], 0, 0)),
        ],
        out_specs=pl.BlockSpec((BM, D), lambda i, te: (i, 0)),
    )
    return pl.pallas_call(
        _gmm_body,
        grid_spec=grid_spec,
        out_shape=jax.ShapeDtypeStruct((S_PAD, D), jnp.float32),
    )(tile_expert, x_pad, expert_w, expert_b.reshape(E, 1, D))


def _combine_body(ypad_hbm, e0_hbm, e1_hbm, r0_hbm, r1_hbm, off_hbm,
                  gate_hbm, out_hbm, off_ref, e_v, r_v,
                  p0_v, p1_v, g0_v, g1_v,
                  a0, a1, b0, b1, gsemA, gsemB, wsemA, wsemB):
    wid = lax.axis_index("s") * NC + lax.axis_index("c")
    base = wid * R_TOK
    pltpu.sync_copy(off_hbm.at[0], off_ref)
    _fill_pos(e0_hbm, r0_hbm, wid, off_ref, e_v, r_v, p0_v, NCH_C, C2)
    _fill_pos(e1_hbm, r1_hbm, wid, off_ref, e_v, r_v, p1_v, NCH_C, C2)
    pltpu.sync_copy(gate_hbm.at[0, wid], g0_v)
    pltpu.sync_copy(gate_hbm.at[1, wid], g1_v)
    pairs = ((a0, a1, gsemA, wsemA), (b0, b1, gsemB, wsemB))
    gath = {0: (pltpu.async_copy(ypad_hbm.at[p0_v.at[0]], a0, gsemA),
                pltpu.async_copy(ypad_hbm.at[p1_v.at[0]], a1, gsemA))}
    writes = {}
    for j in range(NCH_C):
        c0, c1, gsem, wsem = pairs[j % 2]
        gath[j][0].wait()
        gath[j][1].wait()
        if j + 1 < NCH_C:
            n0, n1, ngsem, nwsem = pairs[(j + 1) % 2]
            if j >= 1:
                writes[j - 1].wait()
            gath[j + 1] = (
                pltpu.async_copy(ypad_hbm.at[p0_v.at[j + 1]], n0, ngsem),
                pltpu.async_copy(ypad_hbm.at[p1_v.at[j + 1]], n1, ngsem))

        def row_body(r, _):
            tid = j * C2 + r
            gsl = pl.ds((tid % 8) * L, L)
            gv0 = g0_v[tid // 8, gsl]
            gv1 = g1_v[tid // 8, gsl]

            def vec_body(v, _):
                for u in range(4):
                    sl = pl.ds(v * (4 * L) + u * L, L)
                    c0[r, sl] = c0[r, sl] * gv0 + c1[r, sl] * gv1
                return 0

            lax.fori_loop(0, VG, vec_body, 0)
            return 0

        lax.fori_loop(0, C2, row_body, 0)
        writes[j] = pltpu.async_copy(
            c0, out_hbm.at[pl.ds(base + j * C2, C2)], wsem)
    writes[NCH_C - 2].wait()
    writes[NCH_C - 1].wait()


@functools.cache
def _combine():
    return pl.kernel(
        _combine_body,
        out_type=jax.ShapeDtypeStruct((N_TOK, D), jnp.float32),
        mesh=plsc.VectorSubcoreMesh(core_axis_name="c", subcore_axis_name="s"),
        scratch_types=[
            pltpu.VMEM((L,), jnp.int32),
            pltpu.VMEM((R_TOK // L, L), jnp.int32),
            pltpu.VMEM((R_TOK // L, L), jnp.int32),
            pltpu.VMEM((NCH_C, C2), jnp.int32),
            pltpu.VMEM((NCH_C, C2), jnp.int32),
            pltpu.VMEM((R_TOK // 8, 8 * L), jnp.float32),
            pltpu.VMEM((R_TOK // 8, 8 * L), jnp.float32),
            pltpu.VMEM((C2, D), jnp.float32),
            pltpu.VMEM((C2, D), jnp.float32),
            pltpu.VMEM((C2, D), jnp.float32),
            pltpu.VMEM((C2, D), jnp.float32),
            pltpu.SemaphoreType.DMA,
            pltpu.SemaphoreType.DMA,
            pltpu.SemaphoreType.DMA,
            pltpu.SemaphoreType.DMA,
        ],
    )


def kernel(x, router_w, router_b, expert_w, expert_b):
    idx, gates, ranks, counts8 = _run_router(x, router_w, router_b)
    counts = counts8[0].astype(jnp.int32)

    gp = ((counts + BM - 1) // BM) * BM
    cgp = jnp.cumsum(gp)
    off8 = jnp.broadcast_to((cgp - gp)[None, :], (8, E)).astype(jnp.int32)
    tile_starts = jnp.arange(T_TILES, dtype=jnp.int32)[:, None] * BM
    tile_expert = jnp.minimum(
        jnp.sum(jnp.where(tile_starts >= cgp[None, :], 1, 0), axis=1),
        E - 1).astype(jnp.int32)

    eix = idx.T.reshape(K, NW, R_TOK // L, L)
    rnk = ranks.T.reshape(K, NW, R_TOK // L, L)
    gts = jnp.broadcast_to(
        gates.T.reshape(K, NW, R_TOK, 1),
        (K, NW, R_TOK, L)).reshape(K, NW, R_TOK // 8, 8 * L)
    x_pad = _dispatch()(x, eix[0], eix[1], rnk[0], rnk[1], off8)
    y_pad = _run_gmm(x_pad, expert_w, expert_b, tile_expert)
    return _combine()(y_pad, eix[0], eix[1], rnk[0], rnk[1], off8, gts)


# speedup vs baseline: 1.1191x; 1.0422x over previous
import functools

import jax
import jax.numpy as jnp
from jax import lax
from jax.experimental import pallas as pl
from jax.experimental.pallas import tpu as pltpu
from jax.experimental.pallas import tpu_sc as plsc

N_TOK = 8192
D = 1024
E = 16
K = 2

BM = 256
S = N_TOK * K
S_PAD = S + E * BM
T_TILES = S_PAD // BM

BT = 512
NBT = N_TOK // BT

NC, NS, L = 2, 16, 16
NW = NC * NS
R_TOK = N_TOK // NW

CD = 32
NCH_D = R_TOK // CD

C2 = 16
NCH_C = R_TOK // C2
VG = D // (4 * L)


def _router_body(x_ref, w_ref, b_ref, idx_ref, gates_ref, ranks_ref,
                 counts_ref, carry_ref):
    i = pl.program_id(0)

    @pl.when(i == 0)
    def _():
        carry_ref[...] = jnp.zeros_like(carry_ref)

    xb = x_ref[...]
    logits = jnp.dot(xb, w_ref[...], preferred_element_type=jnp.float32)
    logits = logits + b_ref[0:1, :]

    iota_e = lax.broadcasted_iota(jnp.int32, (BT, E), 1)
    m1 = jnp.max(logits, axis=1, keepdims=True)
    i1 = jnp.min(jnp.where(logits == m1, iota_e, E), axis=1, keepdims=True)
    masked = jnp.where(iota_e == i1, -1e30, logits)
    m2 = jnp.max(masked, axis=1, keepdims=True)
    i2 = jnp.min(jnp.where(masked == m2, iota_e, E), axis=1, keepdims=True)
    g1 = 1.0 / (1.0 + jnp.exp(m2 - m1))
    g2 = 1.0 - g1

    onehot = (jnp.where(iota_e == i1, 1.0, 0.0)
              + jnp.where(iota_e == i2, 1.0, 0.0))
    row_i = lax.broadcasted_iota(jnp.int32, (BT, BT), 0)
    col_i = lax.broadcasted_iota(jnp.int32, (BT, BT), 1)
    tri = jnp.where(row_i > col_i, 1.0, 0.0)
    csum = jnp.dot(tri, onehot, preferred_element_type=jnp.float32)
    ranks_mat = csum + carry_ref[0:1, :]
    r1 = jnp.sum(jnp.where(iota_e == i1, ranks_mat, 0.0), axis=1, keepdims=True)
    r2 = jnp.sum(jnp.where(iota_e == i2, ranks_mat, 0.0), axis=1, keepdims=True)

    carry_new = carry_ref[0:1, :] + jnp.sum(onehot, axis=0, keepdims=True)
    carry_ref[0:1, :] = carry_new

    idx_ref[...] = jnp.concatenate([i1, i2], axis=1)
    gates_ref[...] = jnp.concatenate([g1, g2], axis=1)
    ranks_ref[...] = jnp.concatenate([r1, r2], axis=1).astype(jnp.int32)
    counts_ref[...] = jnp.broadcast_to(carry_new, (8, E))


def _run_router(x, router_w, router_b):
    b2d = jnp.broadcast_to(router_b[None, :], (8, E))
    return pl.pallas_call(
        _router_body,
        grid=(NBT,),
        in_specs=[
            pl.BlockSpec((BT, D), lambda i: (i, 0)),
            pl.BlockSpec((D, E), lambda i: (0, 0)),
            pl.BlockSpec((8, E), lambda i: (0, 0)),
        ],
        out_specs=[
            pl.BlockSpec((BT, K), lambda i: (i, 0)),
            pl.BlockSpec((BT, K), lambda i: (i, 0)),
            pl.BlockSpec((BT, K), lambda i: (i, 0)),
            pl.BlockSpec((8, E), lambda i: (0, 0)),
        ],
        out_shape=[
            jax.ShapeDtypeStruct((N_TOK, K), jnp.int32),
            jax.ShapeDtypeStruct((N_TOK, K), jnp.float32),
            jax.ShapeDtypeStruct((N_TOK, K), jnp.int32),
            jax.ShapeDtypeStruct((8, E), jnp.float32),
        ],
        scratch_shapes=[pltpu.VMEM((8, E), jnp.float32)],
    )(x, router_w, b2d)


def _dispatch_body(x_hbm, pos_hbm, xpad_hbm, p0_v, p1_v, buf0, buf1,
                   rsem0, rsem1, ssem0, ssem1):
    wid = lax.axis_index("s") * NC + lax.axis_index("c")
    base = wid * R_TOK
    reads = {0: pltpu.async_copy(x_hbm.at[pl.ds(base, CD)], buf0, rsem0)}
    pltpu.sync_copy(pos_hbm.at[0, wid], p0_v)
    pltpu.sync_copy(pos_hbm.at[1, wid], p1_v)
    bufs = (buf0, buf1)
    rsems = (rsem0, rsem1)
    ssems = (ssem0, ssem1)
    scats = {}
    for j in range(NCH_D):
        reads[j].wait()
        b = bufs[j % 2]
        s = ssems[j % 2]
        scats[j] = (pltpu.async_copy(b, xpad_hbm.at[p0_v.at[j]], s),
                    pltpu.async_copy(b, xpad_hbm.at[p1_v.at[j]], s))
        if j + 1 < NCH_D:
            if j >= 1:
                scats[j - 1][0].wait()
                scats[j - 1][1].wait()
            reads[j + 1] = pltpu.async_copy(
                x_hbm.at[pl.ds(base + (j + 1) * CD, CD)],
                bufs[(j + 1) % 2], rsems[(j + 1) % 2])
    scats[NCH_D - 2][0].wait()
    scats[NCH_D - 2][1].wait()
    scats[NCH_D - 1][0].wait()
    scats[NCH_D - 1][1].wait()


@functools.cache
def _dispatch():
    return pl.kernel(
        _dispatch_body,
        out_type=jax.ShapeDtypeStruct((S_PAD, D), jnp.float32),
        mesh=plsc.VectorSubcoreMesh(core_axis_name="c", subcore_axis_name="s"),
        scratch_types=[
            pltpu.VMEM((NCH_D, CD), jnp.int32),
            pltpu.VMEM((NCH_D, CD), jnp.int32),
            pltpu.VMEM((CD, D), jnp.float32),
            pltpu.VMEM((CD, D), jnp.float32),
            pltpu.SemaphoreType.DMA,
            pltpu.SemaphoreType.DMA,
            pltpu.SemaphoreType.DMA,
            pltpu.SemaphoreType.DMA,
        ],
    )


def _gmm_body(te_ref, x_ref, w_ref, b_ref, o_ref):
    yb = jnp.dot(x_ref[...], w_ref[0], preferred_element_type=jnp.float32)
    o_ref[...] = yb + b_ref[0]


def _run_gmm(x_pad, expert_w, expert_b, tile_expert):
    grid_spec = pltpu.PrefetchScalarGridSpec(
        num_scalar_prefetch=1,
        grid=(T_TILES,),
        in_specs=[
            pl.BlockSpec((BM, D), lambda i, te: (i, 0)),
            pl.BlockSpec((1, D, D), lambda i, te: (te[i], 0, 0)),
            pl.BlockSpec((1, 1, D), lambda i, te: (te[i], 0, 0)),
        ],
        out_specs=pl.BlockSpec((BM, D), lambda i, te: (i, 0)),
    )
    return pl.pallas_call(
        _gmm_body,
        grid_spec=grid_spec,
        out_shape=jax.ShapeDtypeStruct((S_PAD, D), jnp.float32),
    )(tile_expert, x_pad, expert_w, expert_b.reshape(E, 1, D))


def _combine_body(ypad_hbm, pos_hbm, gate_hbm, out_hbm,
                  p0_v, p1_v, g0_v, g1_v,
                  a0, a1, b0, b1, gsemA, gsemB, wsemA, wsemB):
    wid = lax.axis_index("s") * NC + lax.axis_index("c")
    base = wid * R_TOK
    pltpu.sync_copy(pos_hbm.at[0, wid], p0_v)
    pltpu.sync_copy(pos_hbm.at[1, wid], p1_v)
    pltpu.sync_copy(gate_hbm.at[0, wid], g0_v)
    pltpu.sync_copy(gate_hbm.at[1, wid], g1_v)
    pairs = ((a0, a1, gsemA, wsemA), (b0, b1, gsemB, wsemB))
    gath = {0: (pltpu.async_copy(ypad_hbm.at[p0_v.at[0]], a0, gsemA),
                pltpu.async_copy(ypad_hbm.at[p1_v.at[0]], a1, gsemA))}
    writes = {}
    for j in range(NCH_C):
        c0, c1, gsem, wsem = pairs[j % 2]
        gath[j][0].wait()
        gath[j][1].wait()
        if j + 1 < NCH_C:
            n0, n1, ngsem, nwsem = pairs[(j + 1) % 2]
            if j >= 1:
                writes[j - 1].wait()
            gath[j + 1] = (
                pltpu.async_copy(ypad_hbm.at[p0_v.at[j + 1]], n0, ngsem),
                pltpu.async_copy(ypad_hbm.at[p1_v.at[j + 1]], n1, ngsem))

        def row_body(r, _):
            tid = j * C2 + r
            gsl = pl.ds((tid % 8) * L, L)
            gv0 = g0_v[tid // 8, gsl]
            gv1 = g1_v[tid // 8, gsl]

            def vec_body(v, _):
                for u in range(4):
                    sl = pl.ds(v * (4 * L) + u * L, L)
                    c0[r, sl] = c0[r, sl] * gv0 + c1[r, sl] * gv1
                return 0

            lax.fori_loop(0, VG, vec_body, 0)
            return 0

        lax.fori_loop(0, C2, row_body, 0)
        writes[j] = pltpu.async_copy(
            c0, out_hbm.at[pl.ds(base + j * C2, C2)], wsem)
    writes[NCH_C - 2].wait()
    writes[NCH_C - 1].wait()


@functools.cache
def _combine():
    return pl.kernel(
        _combine_body,
        out_type=jax.ShapeDtypeStruct((N_TOK, D), jnp.float32),
        mesh=plsc.VectorSubcoreMesh(core_axis_name="c", subcore_axis_name="s"),
        scratch_types=[
            pltpu.VMEM((NCH_C, C2), jnp.int32),
            pltpu.VMEM((NCH_C, C2), jnp.int32),
            pltpu.VMEM((R_TOK // 8, 8 * L), jnp.float32),
            pltpu.VMEM((R_TOK // 8, 8 * L), jnp.float32),
            pltpu.VMEM((C2, D), jnp.float32),
            pltpu.VMEM((C2, D), jnp.float32),
            pltpu.VMEM((C2, D), jnp.float32),
            pltpu.VMEM((C2, D), jnp.float32),
            pltpu.SemaphoreType.DMA,
            pltpu.SemaphoreType.DMA,
            pltpu.SemaphoreType.DMA,
            pltpu.SemaphoreType.DMA,
        ],
    )


def kernel(x, router_w, router_b, expert_w, expert_b):
    idx, gates, ranks, counts8 = _run_router(x, router_w, router_b)
    counts = counts8[0].astype(jnp.int32)

    gp = ((counts + BM - 1) // BM) * BM
    cgp = jnp.cumsum(gp)
    off = jnp.concatenate([jnp.zeros((1,), jnp.int32), cgp])
    pos = off[idx] + ranks
    tile_starts = jnp.arange(T_TILES, dtype=jnp.int32)[:, None] * BM
    tile_expert = jnp.minimum(
        jnp.sum(jnp.where(tile_starts >= cgp[None, :], 1, 0), axis=1),
        E - 1).astype(jnp.int32)

    pos_t = pos.T.reshape(K, NW, NCH_D, CD)
    x_pad = _dispatch()(x, pos_t)
    y_pad = _run_gmm(x_pad, expert_w, expert_b, tile_expert)
    pos_c = pos.T.reshape(K, NW, NCH_C, C2)
    gate_b = jnp.broadcast_to(
        gates.T.reshape(K, NW, R_TOK, 1),
        (K, NW, R_TOK, L)).reshape(K, NW, R_TOK // 8, 8 * L)
    return _combine()(y_pad, pos_c, gate_b)
